# baseline (speedup 1.0000x reference)
import functools
import math

import jax
import jax.numpy as jnp
from jax import lax
from jax.experimental import pallas as pl
from jax.experimental.pallas import tpu as pltpu
from jax.experimental.pallas import tpu_sc as plsc

N_NODES = 10000
N_EDGES = 320000
DIMS = 128
NC = 2
NS = 16
NW = NC * NS
CHUNK = 128
NCHUNKS = 80
EPW = NCHUNKS * CHUNK
E_PAD = NW * EPW
N_ACC = 10240
NB = 2
DEG_WIN = 16
RPT = 624
RPT_LAST = N_NODES - 15 * RPT
BN_SCALE = 1.0 / math.sqrt(1.0 + 1e-5)

_mesh = plsc.VectorSubcoreMesh(core_axis_name="c", subcore_axis_name="s")


@functools.partial(
    pl.kernel,
    mesh=_mesh,
    out_type=jax.ShapeDtypeStruct((NC * N_NODES,), jnp.float32),
    scratch_types=[
        pltpu.VMEM((NCHUNKS, CHUNK), jnp.int32),
        pltpu.VMEM((CHUNK,), jnp.float32),
        pltpu.VMEM((N_ACC,), jnp.float32),
        pltpu.VMEM_SHARED((N_ACC,), jnp.float32),
        pltpu.SemaphoreType.DMA,
    ],
)
def _deg_kernel(dst_hbm, zeros_hbm, deg_out, dst_all, ones_v, stage_v,
                deg_sh, sem):
    c = lax.axis_index("c")
    s = lax.axis_index("s")
    w = c * NS + s
    pltpu.sync_copy(dst_hbm.at[pl.ds(w * NCHUNKS, NCHUNKS)], dst_all)
    for j in range(CHUNK // 16):
        ones_v[pl.ds(j * 16, 16)] = jnp.full((16,), 1.0, dtype=jnp.float32)

    @pl.when(s == 0)
    def _init():
        pltpu.sync_copy(zeros_hbm, stage_v)
        pltpu.sync_copy(stage_v, deg_sh)

    plsc.subcore_barrier()

    def body(i, carry):
        @pl.when(i >= DEG_WIN)
        def _throttle():
            pltpu.make_async_copy(ones_v, deg_sh.at[dst_all.at[0]], sem).wait()

        pltpu.async_copy(ones_v, deg_sh.at[dst_all.at[i]], sem, add=True)
        return carry

    lax.fori_loop(0, NCHUNKS, body, 0)

    def drain(i, carry):
        pltpu.make_async_copy(ones_v, deg_sh.at[dst_all.at[0]], sem).wait()
        return carry

    lax.fori_loop(0, DEG_WIN, drain, 0)
    plsc.subcore_barrier()

    @pl.when(s == 0)
    def _drain():
        pltpu.sync_copy(deg_sh.at[pl.ds(0, N_NODES)], stage_v.at[pl.ds(0, N_NODES)])
        pltpu.sync_copy(stage_v.at[pl.ds(0, N_NODES)],
                        deg_out.at[pl.ds(c * N_NODES, N_NODES)])


@functools.partial(
    pl.kernel,
    mesh=_mesh,
    out_type=jax.ShapeDtypeStruct((NC * N_NODES, DIMS), jnp.float32),
    scratch_types=[
        pltpu.VMEM((NCHUNKS, CHUNK), jnp.int32),
        pltpu.VMEM((NB, CHUNK), jnp.int32),
        pltpu.VMEM((NB, CHUNK, DIMS), jnp.float32),
        pltpu.VMEM_SHARED((N_ACC, DIMS), jnp.float32),
        pltpu.SemaphoreType.DMA,
        pltpu.SemaphoreType.DMA,
        pltpu.SemaphoreType.DMA,
    ],
)
def _scatter_kernel(src_hbm, dst_hbm, g_hbm, zrows_hbm, acc_out,
                    src_all, dst_r, rows_v, acc_sh, sem_g, sem_d, sem_s):
    c = lax.axis_index("c")
    s = lax.axis_index("s")
    w = c * NS + s
    pltpu.sync_copy(src_hbm.at[pl.ds(w * NCHUNKS, NCHUNKS)], src_all)

    @pl.when(s < 15)
    def _init_a():
        pltpu.sync_copy(zrows_hbm.at[pl.ds(0, RPT)],
                        acc_sh.at[pl.ds(s * RPT, RPT)])

    @pl.when(s == 15)
    def _init_b():
        pltpu.sync_copy(zrows_hbm, acc_sh.at[pl.ds(15 * RPT, RPT_LAST)])

    plsc.subcore_barrier()

    pltpu.async_copy(g_hbm.at[src_all.at[0]], rows_v.at[0], sem_g)
    pltpu.async_copy(dst_hbm.at[w * NCHUNKS], dst_r.at[0], sem_d)

    def body(i, carry):
        b = lax.rem(i, NB)
        b2 = lax.rem(i + 1, NB)

        @pl.when(i >= 1)
        def _wait_prev_scatter():
            pltpu.make_async_copy(rows_v.at[b2], acc_sh.at[dst_r.at[b2]],
                                  sem_s).wait()

        @pl.when(i + 1 < NCHUNKS)
        def _fire_next():
            pltpu.async_copy(g_hbm.at[src_all.at[i + 1]], rows_v.at[b2], sem_g)
            pltpu.async_copy(dst_hbm.at[w * NCHUNKS + i + 1], dst_r.at[b2],
                             sem_d)

        pltpu.make_async_copy(dst_hbm.at[0], dst_r.at[b], sem_d).wait()
        pltpu.make_async_copy(g_hbm.at[src_all.at[0]], rows_v.at[b],
                              sem_g).wait()
        pltpu.async_copy(rows_v.at[b], acc_sh.at[dst_r.at[b]], sem_s,
                         add=True)
        return carry

    lax.fori_loop(0, NCHUNKS, body, 0)
    pltpu.make_async_copy(rows_v.at[0], acc_sh.at[dst_r.at[0]], sem_s).wait()
    plsc.subcore_barrier()

    @pl.when(s < 15)
    def _drain_a():
        pltpu.sync_copy(acc_sh.at[pl.ds(s * RPT, RPT)],
                        acc_out.at[pl.ds(c * N_NODES + s * RPT, RPT)])

    @pl.when(s == 15)
    def _drain_b():
        pltpu.sync_copy(acc_sh.at[pl.ds(15 * RPT, RPT_LAST)],
                        acc_out.at[pl.ds(c * N_NODES + 15 * RPT, RPT_LAST)])


def _matmul_body(dega_ref, degb_ref, x_ref, w_ref, g_ref):
    deg = dega_ref[...] + degb_ref[...] + 1.0
    dis = lax.rsqrt(deg)
    h = jnp.dot(x_ref[...], w_ref[...], preferred_element_type=jnp.float32)
    g_ref[...] = h * dis


def _epilogue_body(acca_ref, accb_ref, g_ref, dega_ref, degb_ref, x_ref,
                   b_ref, gam_ref, bet_ref, o_ref):
    deg = dega_ref[...] + degb_ref[...] + 1.0
    dis = lax.rsqrt(deg)
    ssum = acca_ref[...] + accb_ref[...] + g_ref[...]
    pre = ssum * dis + b_ref[...]
    bn = pre * (gam_ref[...] * BN_SCALE) + bet_ref[...]
    o_ref[...] = jnp.maximum(bn, 0.0) + x_ref[...]


def kernel(x, edge_index, W, b, gamma, beta):
    n_pad = E_PAD - N_EDGES
    src = edge_index[0].astype(jnp.int32)
    dst = edge_index[1].astype(jnp.int32)
    pad_src = (jnp.arange(n_pad, dtype=jnp.int32) * 13) % N_NODES
    pad_dst = N_NODES + (jnp.arange(n_pad, dtype=jnp.int32) % (N_ACC - N_NODES))
    src2 = jnp.concatenate([src, pad_src]).reshape(NW * NCHUNKS, CHUNK)
    dst2 = jnp.concatenate([dst, pad_dst]).reshape(NW * NCHUNKS, CHUNK)
    zeros1 = jnp.zeros((N_ACC,), jnp.float32)
    zrows = jnp.zeros((RPT_LAST, DIMS), jnp.float32)

    deg2 = _deg_kernel(dst2, zeros1)
    dega = deg2[0:N_NODES].reshape(N_NODES, 1)
    degb = deg2[N_NODES:2 * N_NODES].reshape(N_NODES, 1)

    grid = 10
    br = N_NODES // grid
    g = pl.pallas_call(
        _matmul_body,
        grid=(grid,),
        in_specs=[
            pl.BlockSpec((br, 1), lambda i: (i, 0)),
            pl.BlockSpec((br, 1), lambda i: (i, 0)),
            pl.BlockSpec((br, DIMS), lambda i: (i, 0)),
            pl.BlockSpec((DIMS, DIMS), lambda i: (0, 0)),
        ],
        out_specs=pl.BlockSpec((br, DIMS), lambda i: (i, 0)),
        out_shape=jax.ShapeDtypeStruct((N_NODES, DIMS), jnp.float32),
    )(dega, degb, x, W)

    acc = _scatter_kernel(src2, dst2, g, zrows)

    out = pl.pallas_call(
        _epilogue_body,
        grid=(grid,),
        in_specs=[
            pl.BlockSpec((br, DIMS), lambda i: (i, 0)),
            pl.BlockSpec((br, DIMS), lambda i: (i + grid, 0)),
            pl.BlockSpec((br, DIMS), lambda i: (i, 0)),
            pl.BlockSpec((br, 1), lambda i: (i, 0)),
            pl.BlockSpec((br, 1), lambda i: (i, 0)),
            pl.BlockSpec((br, DIMS), lambda i: (i, 0)),
            pl.BlockSpec((1, DIMS), lambda i: (0, 0)),
            pl.BlockSpec((1, DIMS), lambda i: (0, 0)),
            pl.BlockSpec((1, DIMS), lambda i: (0, 0)),
        ],
        out_specs=pl.BlockSpec((br, DIMS), lambda i: (i, 0)),
        out_shape=jax.ShapeDtypeStruct((N_NODES, DIMS), jnp.float32),
    )(acc, acc, g, dega, degb, x,
      b.reshape(1, DIMS), gamma.reshape(1, DIMS), beta.reshape(1, DIMS))
    return out
